# baseline (speedup 1.0000x reference)
import functools

import jax
import jax.numpy as jnp
import numpy as np
from jax import lax
from jax.experimental import pallas as pl
from jax.experimental.pallas import tpu as pltpu
from jax.experimental.pallas import tpu_sc as plsc

VOCAB = 1000000
D_MODEL = 64
SEQ = 200
NSEQ = 4096

NC = 2
NS = 16
NW = NC * NS

ITERS = NSEQ // NW
UNROLL = 8


def _positional_encoding_np(length, d_model):
    depth = d_model / 2
    depths = np.arange(depth)[np.newaxis, :] / depth
    angle_rads = np.arange(length)[:, np.newaxis] / 10000 ** depths
    return np.concatenate(
        [np.sin(angle_rads), np.cos(angle_rads)], axis=-1
    ).astype(np.float32)


_mesh = plsc.VectorSubcoreMesh(core_axis_name="c", subcore_axis_name="s")


@functools.partial(
    pl.kernel,
    mesh=_mesh,
    out_type=jax.ShapeDtypeStruct((NSEQ, SEQ, D_MODEL), jnp.float32),
    scratch_types=[
        [pltpu.VMEM((SEQ,), jnp.int32)] * 4,
        [pltpu.VMEM((SEQ,), jnp.int32)] * 4,
        [pltpu.VMEM((SEQ, 2 * D_MODEL), jnp.float32)] * 2,
        [pltpu.VMEM((SEQ, D_MODEL), jnp.float32)] * 2,
        pltpu.VMEM((SEQ // 2, 2 * D_MODEL), jnp.float32),
        [pltpu.SemaphoreType.DMA] * 4,
        [pltpu.SemaphoreType.DMA] * 2,
        [pltpu.SemaphoreType.DMA] * 2,
    ],
    compiler_params=pltpu.CompilerParams(use_tc_tiling_on_sc=True),
)
def _emb_kernel(table_hbm, idx_hbm, idx2_hbm, pe_hbm, out_hbm, idx_v, idx2_v,
                rows_v, outc_v, pe_v, sem_i, sem_g, sem_o):
    wid = lax.axis_index("s") * NC + lax.axis_index("c")
    seq0 = wid * ITERS

    pltpu.sync_copy(pe_hbm, pe_v)

    def issue_idx(c, bi):
        pltpu.async_copy(idx_hbm.at[seq0 + c], idx_v[bi], sem_i[bi])
        pltpu.async_copy(idx2_hbm.at[seq0 + c], idx2_v[bi], sem_i[bi])

    SPLITS = ((0, 128), (128, SEQ - 128))

    def prep_gather(c, bi, b):
        pltpu.make_async_copy(idx_hbm.at[seq0], idx_v[bi], sem_i[bi]).wait()
        pltpu.make_async_copy(idx_hbm.at[seq0], idx2_v[bi], sem_i[bi]).wait()
        for off, ln in SPLITS:
            pltpu.async_copy(
                table_hbm.at[idx2_v[bi].at[pl.ds(off, ln)]],
                rows_v[b].at[pl.ds(off, ln)],
                sem_g[b],
            )

    def drain_gather(bi, b):
        for off, ln in SPLITS:
            pltpu.make_async_copy(
                table_hbm.at[idx2_v[bi].at[pl.ds(off, ln)]],
                rows_v[b].at[pl.ds(off, ln)],
                sem_g[b],
            ).wait()

    def issue_out(c, b):
        pltpu.async_copy(outc_v[b], out_hbm.at[seq0 + c], sem_o[b])

    def drain_out(b):
        pltpu.make_async_copy(outc_v[b], out_hbm.at[seq0], sem_o[b]).wait()

    def compute(bi, b):
        def body(i, _):
            base = pl.multiple_of(
                jnp.minimum(i * 16, SEQ - 16).astype(jnp.int32), 8
            )
            ovec = (idx_v[bi][pl.ds(base, 16)] & 1) * D_MODEL
            pbase = lax.shift_right_logical(base, 1)
            for u in range(16):
                r = base + u
                pr = pbase + u // 2
                po = (u % 2) * D_MODEL
                o = pl.multiple_of(ovec[u], D_MODEL)
                for d in range(D_MODEL // 16):
                    src = pl.ds(o + d * 16, 16)
                    dst = pl.ds(d * 16, 16)
                    outc_v[b][r, dst] = (
                        rows_v[b][r, src] * 8.0
                        + pe_v[pr, pl.ds(po + d * 16, 16)]
                    )
            return 0

        lax.fori_loop(0, (SEQ + 15) // 16, body, 0)

    issue_idx(0, 0)
    prep_gather(0, 0, 0)
    issue_idx(1, 1)

    def step(s, _):
        for j in range(4):
            c = 4 * s + j
            bi = j
            b = j % 2
            nbi = (j + 1) % 4
            nb = 1 - b

            drain_gather(bi, b)

            if j < 3:
                prep_gather(c + 1, nbi, nb)
            else:

                @pl.when(s < ITERS // 4 - 1)
                def _():
                    prep_gather(c + 1, nbi, nb)

            compute(bi, b)

            if j < 2:
                issue_idx(c + 2, (j + 2) % 4)
            else:

                @pl.when(s < ITERS // 4 - 1)
                def _():
                    issue_idx(c + 2, (j + 2) % 4)

            if j == 0:

                @pl.when(s > 0)
                def _():
                    drain_out(nb)
            else:
                drain_out(nb)

            issue_out(c, b)

        return 0

    lax.fori_loop(0, ITERS // 4, step, 0)

    drain_out(1)


_PE = _positional_encoding_np(SEQ, D_MODEL)


def kernel(x, table):
    pe = jnp.asarray(_PE.reshape(SEQ // 2, 2 * D_MODEL))
    table2 = table.reshape(VOCAB // 2, 2 * D_MODEL)
    xi = x.astype(jnp.int32)
    return _emb_kernel(table2, xi, xi >> 1, pe)
